# baseline (speedup 1.0000x reference)
import functools

import jax
import jax.numpy as jnp
from jax import lax
from jax.experimental import pallas as pl
from jax.experimental.pallas import tpu as pltpu
from jax.experimental.pallas import tpu_sc as plsc

CHUNK = 512
GROUPS = CHUNK // 16
NC = 2
NS = 16
NW = NC * NS
STG = 8192

ATAN_C = (
    0.9999999999999999, -0.33333332887532793, 0.19999993617905452,
    -0.14285569384178593, 0.11110479717764981, -0.09087862725762335,
    0.07674740373359167, -0.06445430808694534, 0.05136716521978941,
    -0.0337415742387042, 0.010436832014272276,
)
HALF_PI = 1.5707963267948966
RSQRT_MAGIC = 0x5F3759DF


def _rsqrt(x):
    i = lax.bitcast_convert_type(x, jnp.int32)
    i = RSQRT_MAGIC - lax.shift_right_logical(i, 1)
    y = lax.bitcast_convert_type(i, jnp.float32)
    for _ in range(3):
        y = y * (1.5 - 0.5 * x * y * y)
    return y


def _atan01(x):
    u = x * x
    p = jnp.full_like(x, ATAN_C[-1])
    for c in ATAN_C[-2::-1]:
        p = p * u + c
    return p * x


def _qmul(ax, ay, az, aw, bx, by, bz, bw):
    x = aw * bx + ax * bw + ay * bz - az * by
    y = aw * by - ax * bz + ay * bw + az * bx
    z = aw * bz + ax * by - ay * bx + az * bw
    w = aw * bw - ax * bx - ay * by - az * bz
    return x, y, z, w


def _qrot(qx, qy, qz, qw, vx, vy, vz):
    tx = 2.0 * (qy * vz - qz * vy)
    ty = 2.0 * (qz * vx - qx * vz)
    tz = 2.0 * (qx * vy - qy * vx)
    rx = vx + qw * tx + qy * tz - qz * ty
    ry = vy + qw * ty + qz * tx - qx * tz
    rz = vz + qw * tz + qx * ty - qy * tx
    return rx, ry, rz


def _se3_err_log(n1, n2, p):
    t1x, t1y, t1z, q1x, q1y, q1z, q1w = n1
    t2x, t2y, t2z, q2x, q2y, q2z, q2w = n2
    tpx, tpy, tpz, qpx, qpy, qpz, qpw = p
    ax, ay, az, aw = -q1x, -q1y, -q1z, q1w
    mx, my, mz, mw = _qmul(ax, ay, az, aw, q2x, q2y, q2z, q2w)
    ex, ey, ez, ew = _qmul(mx, my, mz, mw, -qpx, -qpy, -qpz, qpw)
    ux, uy, uz = _qrot(ax, ay, az, aw, t2x - t1x, t2y - t1y, t2z - t1z)
    vx, vy, vz = _qrot(ex, ey, ez, ew, tpx, tpy, tpz)
    tx, ty, tz = ux - vx, uy - vy, uz - vz
    s = jnp.where(ew < 0.0, -1.0, 1.0)
    qx, qy, qz, qw = s * ex, s * ey, s * ez, s * ew
    nn = qx * qx + qy * qy + qz * qz
    n2p = nn + 1e-24
    inv_n = _rsqrt(n2p)
    n = n2p * inv_n
    hi = jnp.maximum(n, qw)
    lo = jnp.minimum(n, qw)
    a = _atan01(lo / hi)
    theta = 2.0 * jnp.where(n > qw, HALF_PI - a, a)
    small = n < 1e-6
    f = jnp.where(small, 2.0 / jnp.maximum(qw, 1e-6), theta * inv_n)
    px, py, pz = f * qx, f * qy, f * qz
    den = 2.0 * theta * theta * n
    coef = jnp.where(small, 1.0 / 12.0,
                     (2.0 * n - theta * qw) / jnp.where(small, 1.0, den))
    cx = py * tz - pz * ty
    cy = pz * tx - px * tz
    cz = px * ty - py * tx
    dx = py * cz - pz * cy
    dy = pz * cx - px * cz
    dz = px * cy - py * cx
    taux = tx - 0.5 * cx + coef * dx
    tauy = ty - 0.5 * cy + coef * dy
    tauz = tz - 0.5 * cz + coef * dz
    return taux, tauy, tauz, px, py, pz


def _make_kernel(E, N):
    assert E % CHUNK == 0
    nchunks = E // CHUNK
    niter = (nchunks + NW - 1) // NW
    nwords = N * 8
    span = (nwords // (NS * 8)) * 8
    rem = nwords - NS * span
    mesh = plsc.VectorSubcoreMesh(core_axis_name="c", subcore_axis_name="s")

    @functools.partial(
        pl.kernel,
        mesh=mesh,
        compiler_params=pltpu.CompilerParams(needs_layout_passes=False),
        out_type=jax.ShapeDtypeStruct((E * 6,), jnp.float32),
        scratch_types=[
            pltpu.VMEM((CHUNK,), jnp.int32),
            pltpu.VMEM((CHUNK,), jnp.int32),
        ] + [pltpu.VMEM((CHUNK,), jnp.int32) for _ in range(14)
        ] + [pltpu.VMEM((CHUNK,), jnp.float32) for _ in range(14)
        ] + [
            pltpu.VMEM((CHUNK * 7,), jnp.float32),
            pltpu.VMEM((CHUNK * 6,), jnp.float32),
            pltpu.VMEM((STG,), jnp.float32),
            pltpu.VMEM_SHARED((N * 8,), jnp.float32),
            pltpu.SemaphoreType.DMA,
            pltpu.SemaphoreType.DMA,
        ],
    )
    def k(idx0_hbm, idx1_hbm, poses_hbm, nodes_hbm, out_hbm,
          i0v, i1v, *scr):
        w0 = scr[0:7]
        w1 = scr[7:14]
        c0 = scr[14:21]
        c1 = scr[21:28]
        pv, ov, stg, sh, sem0, sem1 = scr[28:]
        sid = lax.axis_index("s")
        wid = sid * NC + lax.axis_index("c")
        lanes = lax.iota(jnp.int32, 16)

        base_w = sid * span
        for s0 in range(0, span, STG):
            sz = min(STG, span - s0)
            pltpu.sync_copy(nodes_hbm.at[pl.ds(base_w + s0, sz)],
                            stg.at[pl.ds(0, sz)])
            pltpu.sync_copy(stg.at[pl.ds(0, sz)],
                            sh.at[pl.ds(base_w + s0, sz)])
        if rem:
            @pl.when(sid == 0)
            def _():
                rb = NS * span
                pltpu.sync_copy(nodes_hbm.at[pl.ds(rb, rem)],
                                stg.at[pl.ds(0, rem)])
                pltpu.sync_copy(stg.at[pl.ds(0, rem)],
                                sh.at[pl.ds(rb, rem)])
        plsc.subcore_barrier()

        def body(i, carry):
            c = wid + i * NW

            @pl.when(c < nchunks)
            def _():
                base = c * CHUNK
                pltpu.sync_copy(idx0_hbm.at[pl.ds(base, CHUNK)], i0v)
                pltpu.sync_copy(idx1_hbm.at[pl.ds(base, CHUNK)], i1v)
                for g in range(GROUPS):
                    sl = pl.ds(g * 16, 16)
                    v0 = i0v[sl] * 8
                    v1 = i1v[sl] * 8
                    for j in range(7):
                        w0[j][sl] = v0 + j
                        w1[j][sl] = v1 + j
                cps = []
                for j in range(7):
                    cp = pltpu.make_async_copy(sh.at[w0[j]], c0[j], sem0)
                    cp.start()
                    cps.append(cp)
                for j in range(7):
                    cp = pltpu.make_async_copy(sh.at[w1[j]], c1[j], sem1)
                    cp.start()
                    cps.append(cp)
                pltpu.sync_copy(poses_hbm.at[pl.ds(base * 7, CHUNK * 7)], pv)
                for cp in cps:
                    cp.wait()
                for g in range(GROUPS):
                    sl = pl.ds(g * 16, 16)
                    rows = lanes + (g * 16)
                    node1 = tuple(c0[j][sl] for j in range(7))
                    node2 = tuple(c1[j][sl] for j in range(7))
                    pose = tuple(plsc.load_gather(pv, [rows * 7 + j])
                                 for j in range(7))
                    res = _se3_err_log(node1, node2, pose)
                    for j in range(6):
                        plsc.store_scatter(ov, [rows * 6 + j], res[j])
                pltpu.sync_copy(ov, out_hbm.at[pl.ds(base * 6, CHUNK * 6)])
            return carry

        lax.fori_loop(0, niter, body, 0)

    return k


def kernel(edges, poses, nodes):
    E = edges.shape[0]
    N = nodes.shape[0]
    edges = edges.astype(jnp.int32)
    idx0 = edges[:, 0]
    idx1 = edges[:, 1]
    poses_flat = poses.reshape(-1)
    nodes_flat = jnp.pad(nodes, ((0, 0), (0, 1))).reshape(-1)
    out = _make_kernel(E, N)(idx0, idx1, poses_flat, nodes_flat)
    return out.reshape(E, 6)


# speedup vs baseline: 1.1926x; 1.1926x over previous
import functools

import jax
import jax.numpy as jnp
from jax import lax
from jax.experimental import pallas as pl
from jax.experimental.pallas import tpu as pltpu
from jax.experimental.pallas import tpu_sc as plsc

CHUNK = 128
GROUPS = CHUNK // 16
NC = 2
NS = 16
NW = NC * NS
STG = 8192

ATAN_C = (
    0.9999999999999999, -0.33333332887532793, 0.19999993617905452,
    -0.14285569384178593, 0.11110479717764981, -0.09087862725762335,
    0.07674740373359167, -0.06445430808694534, 0.05136716521978941,
    -0.0337415742387042, 0.010436832014272276,
)
HALF_PI = 1.5707963267948966
RSQRT_MAGIC = 0x5F3759DF


def _rsqrt(x):
    i = lax.bitcast_convert_type(x, jnp.int32)
    i = RSQRT_MAGIC - lax.shift_right_logical(i, 1)
    y = lax.bitcast_convert_type(i, jnp.float32)
    for _ in range(3):
        y = y * (1.5 - 0.5 * x * y * y)
    return y


def _atan01(x):
    u = x * x
    p = jnp.full_like(x, ATAN_C[-1])
    for c in ATAN_C[-2::-1]:
        p = p * u + c
    return p * x


def _qmul(ax, ay, az, aw, bx, by, bz, bw):
    x = aw * bx + ax * bw + ay * bz - az * by
    y = aw * by - ax * bz + ay * bw + az * bx
    z = aw * bz + ax * by - ay * bx + az * bw
    w = aw * bw - ax * bx - ay * by - az * bz
    return x, y, z, w


def _qrot(qx, qy, qz, qw, vx, vy, vz):
    tx = 2.0 * (qy * vz - qz * vy)
    ty = 2.0 * (qz * vx - qx * vz)
    tz = 2.0 * (qx * vy - qy * vx)
    rx = vx + qw * tx + qy * tz - qz * ty
    ry = vy + qw * ty + qz * tx - qx * tz
    rz = vz + qw * tz + qx * ty - qy * tx
    return rx, ry, rz


def _se3_err_log(n1, n2, p):
    t1x, t1y, t1z, q1x, q1y, q1z, q1w = n1
    t2x, t2y, t2z, q2x, q2y, q2z, q2w = n2
    tpx, tpy, tpz, qpx, qpy, qpz, qpw = p
    ax, ay, az, aw = -q1x, -q1y, -q1z, q1w
    mx, my, mz, mw = _qmul(ax, ay, az, aw, q2x, q2y, q2z, q2w)
    ex, ey, ez, ew = _qmul(mx, my, mz, mw, -qpx, -qpy, -qpz, qpw)
    ux, uy, uz = _qrot(ax, ay, az, aw, t2x - t1x, t2y - t1y, t2z - t1z)
    vx, vy, vz = _qrot(ex, ey, ez, ew, tpx, tpy, tpz)
    tx, ty, tz = ux - vx, uy - vy, uz - vz
    s = jnp.where(ew < 0.0, -1.0, 1.0)
    qx, qy, qz, qw = s * ex, s * ey, s * ez, s * ew
    nn = qx * qx + qy * qy + qz * qz
    n2p = nn + 1e-24
    inv_n = _rsqrt(n2p)
    n = n2p * inv_n
    hi = jnp.maximum(n, qw)
    lo = jnp.minimum(n, qw)
    a = _atan01(lo / hi)
    theta = 2.0 * jnp.where(n > qw, HALF_PI - a, a)
    small = n < 1e-6
    f = jnp.where(small, 2.0 / jnp.maximum(qw, 1e-6), theta * inv_n)
    px, py, pz = f * qx, f * qy, f * qz
    den = 2.0 * theta * theta * n
    coef = jnp.where(small, 1.0 / 12.0,
                     (2.0 * n - theta * qw) / jnp.where(small, 1.0, den))
    cx = py * tz - pz * ty
    cy = pz * tx - px * tz
    cz = px * ty - py * tx
    dx = py * cz - pz * cy
    dy = pz * cx - px * cz
    dz = px * cy - py * cx
    taux = tx - 0.5 * cx + coef * dx
    tauy = ty - 0.5 * cy + coef * dy
    tauz = tz - 0.5 * cz + coef * dz
    return taux, tauy, tauz, px, py, pz


def _make_kernel(E, N):
    assert E % CHUNK == 0
    nchunks = E // CHUNK
    niter = (nchunks + NW - 1) // NW
    nwords = N * 8
    span = (nwords // (NS * 8)) * 8
    rem = nwords - NS * span
    mesh = plsc.VectorSubcoreMesh(core_axis_name="c", subcore_axis_name="s")

    @functools.partial(
        pl.kernel,
        mesh=mesh,
        compiler_params=pltpu.CompilerParams(needs_layout_passes=False),
        out_type=jax.ShapeDtypeStruct((E * 6,), jnp.float32),
        scratch_types=(
            [pltpu.VMEM((CHUNK,), jnp.int32) for _ in range(4)]
            + [pltpu.VMEM((CHUNK,), jnp.int32) for _ in range(28)]
            + [pltpu.VMEM((CHUNK,), jnp.float32) for _ in range(28)]
            + [pltpu.VMEM((CHUNK * 7,), jnp.float32) for _ in range(2)]
            + [pltpu.VMEM((CHUNK * 6,), jnp.float32) for _ in range(2)]
            + [pltpu.VMEM((STG,), jnp.float32)]
            + [pltpu.VMEM_SHARED((N * 8,), jnp.float32)]
            + [pltpu.SemaphoreType.DMA for _ in range(10)]
        ),
    )
    def k(idx0_hbm, idx1_hbm, poses_hbm, nodes_hbm, out_hbm, *scr):
        i0v = scr[0:2]
        i1v = scr[2:4]
        w0 = [scr[4 + b * 14:4 + b * 14 + 7] for b in range(2)]
        w1 = [scr[11 + b * 14:11 + b * 14 + 7] for b in range(2)]
        c0 = [scr[32 + b * 14:32 + b * 14 + 7] for b in range(2)]
        c1 = [scr[39 + b * 14:39 + b * 14 + 7] for b in range(2)]
        pv = scr[60:62]
        ov = scr[62:64]
        stg = scr[64]
        sh = scr[65]
        sem_i = scr[66:68]
        sem_g0 = scr[68:70]
        sem_g1 = scr[70:72]
        sem_p = scr[72:74]
        sem_o = scr[74:76]
        sid = lax.axis_index("s")
        wid = sid * NC + lax.axis_index("c")
        lanes = lax.iota(jnp.int32, 16)

        base_w = sid * span
        for s0 in range(0, span, STG):
            sz = min(STG, span - s0)
            pltpu.sync_copy(nodes_hbm.at[pl.ds(base_w + s0, sz)],
                            stg.at[pl.ds(0, sz)])
            pltpu.sync_copy(stg.at[pl.ds(0, sz)],
                            sh.at[pl.ds(base_w + s0, sz)])
        if rem:
            @pl.when(sid == 0)
            def _():
                rb = NS * span
                pltpu.sync_copy(nodes_hbm.at[pl.ds(rb, rem)],
                                stg.at[pl.ds(0, rem)])
                pltpu.sync_copy(stg.at[pl.ds(0, rem)],
                                sh.at[pl.ds(rb, rem)])
        plsc.subcore_barrier()

        def fire_idx(cn, b):
            bs = cn * CHUNK
            pltpu.make_async_copy(
                idx0_hbm.at[pl.ds(bs, CHUNK)], i0v[b], sem_i[b]).start()
            pltpu.make_async_copy(
                idx1_hbm.at[pl.ds(bs, CHUNK)], i1v[b], sem_i[b]).start()

        def fire_pose(cn, b):
            bs = cn * CHUNK
            pltpu.make_async_copy(
                poses_hbm.at[pl.ds(bs * 7, CHUNK * 7)], pv[b], sem_p[b]).start()

        def wait_idx(b):
            pltpu.make_async_copy(
                idx0_hbm.at[pl.ds(0, CHUNK)], i0v[b], sem_i[b]).wait()
            pltpu.make_async_copy(
                idx1_hbm.at[pl.ds(0, CHUNK)], i1v[b], sem_i[b]).wait()

        def fire_gathers(b):
            for g in range(GROUPS):
                sl = pl.ds(g * 16, 16)
                v0 = i0v[b][sl] * 8
                v1 = i1v[b][sl] * 8
                for j in range(7):
                    w0[b][j][sl] = v0 + j
                    w1[b][j][sl] = v1 + j
            for j in range(7):
                pltpu.make_async_copy(
                    sh.at[w0[b][j]], c0[b][j], sem_g0[b]).start()
                pltpu.make_async_copy(
                    sh.at[w1[b][j]], c1[b][j], sem_g1[b]).start()

        def wait_gathers(b):
            for j in range(7):
                pltpu.make_async_copy(
                    sh.at[w0[b][j]], c0[b][j], sem_g0[b]).wait()
                pltpu.make_async_copy(
                    sh.at[w1[b][j]], c1[b][j], sem_g1[b]).wait()

        def wait_pose(b):
            pltpu.make_async_copy(
                poses_hbm.at[pl.ds(0, CHUNK * 7)], pv[b], sem_p[b]).wait()

        def wait_out(b, bs):
            pltpu.make_async_copy(
                ov[b], out_hbm.at[pl.ds(bs, CHUNK * 6)], sem_o[b]).wait()

        def compute(b, c):
            for g in range(GROUPS):
                sl = pl.ds(g * 16, 16)
                rows = lanes + (g * 16)
                node1 = tuple(c0[b][j][sl] for j in range(7))
                node2 = tuple(c1[b][j][sl] for j in range(7))
                pose = tuple(plsc.load_gather(pv[b], [rows * 7 + j])
                             for j in range(7))
                res = _se3_err_log(node1, node2, pose)
                for j in range(6):
                    plsc.store_scatter(ov[b], [rows * 6 + j], res[j])
            pltpu.make_async_copy(
                ov[b], out_hbm.at[pl.ds(c * CHUNK * 6, CHUNK * 6)],
                sem_o[b]).start()

        fire_idx(wid, 0)
        fire_idx(wid + NW, 1)
        fire_pose(wid, 0)
        wait_idx(0)
        fire_gathers(0)

        def body(i, carry):
            b = lax.rem(i, 2)
            c = wid + i * NW

            for bb in range(2):
                @pl.when((b == bb) & (c < nchunks))
                def _():
                    wait_gathers(bb)
                    wait_pose(bb)
                    bn = 1 - bb

                    @pl.when(c + NW < nchunks)
                    def _():
                        wait_idx(bn)
                        fire_gathers(bn)
                        fire_pose(c + NW, bn)

                    @pl.when(c + 2 * NW < nchunks)
                    def _():
                        fire_idx(c + 2 * NW, bb)

                    @pl.when(i >= 2)
                    def _():
                        wait_out(bb, c * CHUNK * 6)
                    compute(bb, c)
            return carry

        lax.fori_loop(0, niter, body, 0)
        wait_out(0, 0)
        wait_out(1, 0)

    return k


def kernel(edges, poses, nodes):
    E = edges.shape[0]
    N = nodes.shape[0]
    edges = edges.astype(jnp.int32)
    idx0 = edges[:, 0]
    idx1 = edges[:, 1]
    poses_flat = poses.reshape(-1)
    nodes_flat = jnp.pad(nodes, ((0, 0), (0, 1))).reshape(-1)
    out = _make_kernel(E, N)(idx0, idx1, poses_flat, nodes_flat)
    return out.reshape(E, 6)
